# baseline (speedup 1.0000x reference)
import jax
import jax.numpy as jnp
from jax import lax
from jax.experimental import pallas as pl
from jax.experimental.pallas import tpu as pltpu

_N = 4096
_RB = 256
_NBLK = _N // _RB
_TOPK = 31
_BITS = 34
_INTERPRET = False


def _sortable(x_f32):
    b = lax.bitcast_convert_type(x_f32, jnp.int32)
    return jnp.where(b >= 0, b, b ^ jnp.int32(0x7FFFFFFF))


def _unsortable(key_i32):
    b = jnp.where(key_i32 >= 0, key_i32, key_i32 ^ jnp.int32(0x7FFFFFFF))
    return lax.bitcast_convert_type(b, jnp.float32)


def _k1_body(f_ref, x_ref, wg0_ref, bg0_ref, wg1_ref, bg1_ref, wc0_ref,
             emb_ref, xw_ref):
    h = lax.dot_general(f_ref[...], wg0_ref[...], (((1,), (1,)), ((), ())))
    h = jnp.maximum(h + bg0_ref[...], 0.0)
    h = lax.dot_general(h, wg1_ref[...], (((1,), (1,)), ((), ()))) \
        + bg1_ref[...]
    nrm = jnp.sqrt(jnp.sum(h * h, axis=1, keepdims=True))
    emb_ref[...] = h / jnp.maximum(nrm, 1e-12)
    xw_ref[...] = lax.dot_general(x_ref[...], wc0_ref[...],
                                  (((1,), (0,)), ((), ())))


def _k2_body(emb_blk_ref, emb_full_ref, t_ref, rs_ref, cs_ref, key_scr):
    i = pl.program_id(0)
    raw = lax.dot_general(emb_blk_ref[...], emb_full_ref[...],
                          (((1,), (1,)), ((), ())))
    key_scr[...] = _sortable(raw)

    lo0 = jnp.full((_RB, 1), _sortable(jnp.float32(-2.0)), jnp.int32)
    hi0 = jnp.full((_RB, 1), _sortable(jnp.float32(2.0)), jnp.int32)

    def cond(carry):
        it, done, _, _ = carry
        return jnp.logical_and(it < _BITS, jnp.logical_not(done))

    def body(carry):
        it, _, lo, hi = carry
        mid = (lo >> 1) + (hi >> 1) + (lo & hi & 1)
        k = key_scr[...]
        cnt = jnp.sum(jnp.where(k >= mid, 1, 0).astype(jnp.int32),
                      axis=1, keepdims=True)
        ge = cnt >= _TOPK
        lo = jnp.where(ge, mid, lo)
        hi = jnp.where(ge, hi, mid)
        done = jnp.all(cnt == _TOPK)
        return it + 1, done, lo, hi

    _, _, lo, _ = lax.while_loop(cond, body, (0, False, lo0, hi0))
    t = _unsortable(lo)
    t_ref[...] = t
    raw2 = _unsortable(key_scr[...])
    masked = jnp.where(raw2 >= t, jnp.maximum(raw2, 0.0), 0.0)
    rs_ref[...] = jnp.sum(masked, axis=1, keepdims=True)
    part = jnp.sum(masked, axis=0, keepdims=True)

    @pl.when(i == 0)
    def _():
        cs_ref[...] = part

    @pl.when(i != 0)
    def _():
        cs_ref[...] = cs_ref[...] + part


def _adjn_block(emb_blk_ref, emb_full_ref, tcol_ref, trow_ref, dcol_ref,
                drow_ref):
    raw = lax.dot_general(emb_blk_ref[...], emb_full_ref[...],
                          (((1,), (1,)), ((), ())))
    rp = jnp.maximum(raw, 0.0)
    msum = (jnp.where(raw >= tcol_ref[...], 0.5, 0.0)
            + jnp.where(raw >= trow_ref[...], 0.5, 0.0))
    return (dcol_ref[...] * (rp * msum)) * drow_ref[...]


def _k3_body(emb_blk_ref, emb_full_ref, tcol_ref, trow_ref, dcol_ref,
             drow_ref, xw_ref, wc1_ref, bc0_ref, h1w_ref):
    adjn = _adjn_block(emb_blk_ref, emb_full_ref, tcol_ref, trow_ref,
                       dcol_ref, drow_ref)
    h1 = lax.dot_general(adjn, xw_ref[...], (((1,), (0,)), ((), ()))) \
        + bc0_ref[...]
    h1 = jnp.maximum(h1, 0.0)
    h1w_ref[...] = lax.dot_general(h1, wc1_ref[...], (((1,), (0,)), ((), ())))


def _k4_body(emb_blk_ref, emb_full_ref, tcol_ref, trow_ref, dcol_ref,
             drow_ref, h1w_ref, bc1_ref, adjn_ref, out_ref):
    adjn = _adjn_block(emb_blk_ref, emb_full_ref, tcol_ref, trow_ref,
                       dcol_ref, drow_ref)
    adjn_ref[...] = adjn
    out_ref[...] = lax.dot_general(adjn, h1w_ref[...],
                                   (((1,), (0,)), ((), ()))) + bc1_ref[...]


def kernel(features, x, W_g0, b_g0, W_g1, b_g1, W_c0, b_c0, W_c1, b_c1):
    f32 = jnp.float32
    n, din = features.shape
    hid = W_c0.shape[1]
    dout = W_c1.shape[1]
    bg0 = b_g0.reshape(1, -1)
    bg1 = b_g1.reshape(1, -1)
    bc0 = b_c0.reshape(1, -1)
    bc1 = b_c1.reshape(1, -1)

    full = lambda s: pl.BlockSpec(s, lambda i: (0, 0))
    rows = lambda c: pl.BlockSpec((_RB, c), lambda i: (i, 0))
    col1 = pl.BlockSpec((_RB, 1), lambda i: (i, 0))

    emb, xw = pl.pallas_call(
        _k1_body,
        grid=(_NBLK,),
        in_specs=[rows(din), rows(din), full((din, din)), full((1, din)),
                  full((din, din)), full((1, din)), full((din, hid))],
        out_specs=[rows(din), rows(hid)],
        out_shape=[jax.ShapeDtypeStruct((n, din), f32),
                   jax.ShapeDtypeStruct((n, hid), f32)],
        interpret=_INTERPRET,
    )(features, x, W_g0, bg0, W_g1, bg1, W_c0)

    t, rs, cs = pl.pallas_call(
        _k2_body,
        grid=(_NBLK,),
        in_specs=[rows(din), full((n, din))],
        out_specs=[col1, col1, pl.BlockSpec((1, n), lambda i: (0, 0))],
        out_shape=[jax.ShapeDtypeStruct((n, 1), f32),
                   jax.ShapeDtypeStruct((n, 1), f32),
                   jax.ShapeDtypeStruct((1, n), f32)],
        scratch_shapes=[pltpu.VMEM((_RB, n), jnp.int32)],
        interpret=_INTERPRET,
    )(emb, emb)

    deg = (rs + cs.reshape(n, 1)) * 0.5
    d = 1.0 / (jnp.sqrt(deg) + 1e-10)
    t_row = t.reshape(1, n)
    d_row = d.reshape(1, n)

    h1w = pl.pallas_call(
        _k3_body,
        grid=(_NBLK,),
        in_specs=[rows(din), full((n, din)), col1, full((1, n)), col1,
                  full((1, n)), full((n, hid)), full((hid, dout)),
                  full((1, hid))],
        out_specs=rows(dout),
        out_shape=jax.ShapeDtypeStruct((n, dout), f32),
        interpret=_INTERPRET,
    )(emb, emb, t, t_row, d, d_row, xw, W_c1, bc0)

    adjn, out = pl.pallas_call(
        _k4_body,
        grid=(_NBLK,),
        in_specs=[rows(din), full((n, din)), col1, full((1, n)), col1,
                  full((1, n)), full((n, dout)), full((1, dout))],
        out_specs=[rows(n), rows(dout)],
        out_shape=[jax.ShapeDtypeStruct((n, n), f32),
                   jax.ShapeDtypeStruct((n, dout), f32)],
        interpret=_INTERPRET,
    )(emb, emb, t, t_row, d, d_row, h1w, bc1)

    return (out, adjn)


# speedup vs baseline: 1.1170x; 1.1170x over previous
import jax
import jax.numpy as jnp
from jax import lax
from jax.experimental import pallas as pl
from jax.experimental.pallas import tpu as pltpu

_N = 4096
_RB = 256
_NBLK = _N // _RB
_TOPK = 31
_BITS = 34
_INTERPRET = False


def _sortable(x_f32):
    b = lax.bitcast_convert_type(x_f32, jnp.int32)
    return jnp.where(b >= 0, b, b ^ jnp.int32(0x7FFFFFFF))


def _unsortable(key_i32):
    b = jnp.where(key_i32 >= 0, key_i32, key_i32 ^ jnp.int32(0x7FFFFFFF))
    return lax.bitcast_convert_type(b, jnp.float32)


def _k1_body(f_ref, x_ref, wg0_ref, bg0_ref, wg1_ref, bg1_ref, wc0_ref,
             emb_ref, xw_ref):
    h = lax.dot_general(f_ref[...], wg0_ref[...], (((1,), (1,)), ((), ())))
    h = jnp.maximum(h + bg0_ref[...], 0.0)
    h = lax.dot_general(h, wg1_ref[...], (((1,), (1,)), ((), ()))) \
        + bg1_ref[...]
    nrm = jnp.sqrt(jnp.sum(h * h, axis=1, keepdims=True))
    emb_ref[...] = h / jnp.maximum(nrm, 1e-12)
    xw_ref[...] = lax.dot_general(x_ref[...], wc0_ref[...],
                                  (((1,), (0,)), ((), ())))


def _k2_body(emb_blk_ref, emb_full_ref, t_ref, rs_ref, cs_ref, key_scr):
    i = pl.program_id(0)
    raw = lax.dot_general(emb_blk_ref[...], emb_full_ref[...],
                          (((1,), (1,)), ((), ())))
    key_scr[...] = _sortable(raw)

    lo0 = jnp.full((_RB, 1), _sortable(jnp.float32(-2.0)), jnp.int32)
    hi0 = jnp.full((_RB, 1), _sortable(jnp.float32(2.0)), jnp.int32)

    def cond(carry):
        it, done, _, _, _ = carry
        return jnp.logical_and(it < _BITS, jnp.logical_not(done))

    def body(carry):
        it, _, conv, lo, hi = carry
        mid = (lo >> 1) + (hi >> 1) + (lo & hi & 1)
        k = key_scr[...]
        cnt = jnp.sum(jnp.where(k >= mid, 1, 0).astype(jnp.int32),
                      axis=1, keepdims=True)
        ge = cnt >= _TOPK
        lo = jnp.where(ge, mid, lo)
        hi = jnp.where(ge, hi, mid)
        conv = jnp.maximum(conv, jnp.where(cnt == _TOPK, 1, 0))
        return it + 1, jnp.min(conv) >= 1, conv, lo, hi

    conv0 = jnp.zeros((_RB, 1), jnp.int32)
    _, _, _, lo, _ = lax.while_loop(cond, body, (0, False, conv0, lo0, hi0))
    t = _unsortable(lo)
    t_ref[...] = t
    raw2 = _unsortable(key_scr[...])
    masked = jnp.where(raw2 >= t, jnp.maximum(raw2, 0.0), 0.0)
    rs_ref[...] = jnp.sum(masked, axis=1, keepdims=True)
    part = jnp.sum(masked, axis=0, keepdims=True)

    @pl.when(i == 0)
    def _():
        cs_ref[...] = part

    @pl.when(i != 0)
    def _():
        cs_ref[...] = cs_ref[...] + part


def _adjn_block(emb_blk_ref, emb_full_ref, tcol_ref, trow_ref, dcol_ref,
                drow_ref):
    raw = lax.dot_general(emb_blk_ref[...], emb_full_ref[...],
                          (((1,), (1,)), ((), ())))
    rp = jnp.maximum(raw, 0.0)
    msum = (jnp.where(raw >= tcol_ref[...], 0.5, 0.0)
            + jnp.where(raw >= trow_ref[...], 0.5, 0.0))
    return (dcol_ref[...] * (rp * msum)) * drow_ref[...]


def _k3_body(emb_blk_ref, emb_full_ref, tcol_ref, trow_ref, dcol_ref,
             drow_ref, xw_ref, wc1_ref, bc0_ref, h1w_ref):
    adjn = _adjn_block(emb_blk_ref, emb_full_ref, tcol_ref, trow_ref,
                       dcol_ref, drow_ref)
    h1 = lax.dot_general(adjn, xw_ref[...], (((1,), (0,)), ((), ()))) \
        + bc0_ref[...]
    h1 = jnp.maximum(h1, 0.0)
    h1w_ref[...] = lax.dot_general(h1, wc1_ref[...], (((1,), (0,)), ((), ())))


def _k4_body(emb_blk_ref, emb_full_ref, tcol_ref, trow_ref, dcol_ref,
             drow_ref, h1w_ref, bc1_ref, adjn_ref, out_ref):
    adjn = _adjn_block(emb_blk_ref, emb_full_ref, tcol_ref, trow_ref,
                       dcol_ref, drow_ref)
    adjn_ref[...] = adjn
    out_ref[...] = lax.dot_general(adjn, h1w_ref[...],
                                   (((1,), (0,)), ((), ()))) + bc1_ref[...]


def kernel(features, x, W_g0, b_g0, W_g1, b_g1, W_c0, b_c0, W_c1, b_c1):
    f32 = jnp.float32
    n, din = features.shape
    hid = W_c0.shape[1]
    dout = W_c1.shape[1]
    bg0 = b_g0.reshape(1, -1)
    bg1 = b_g1.reshape(1, -1)
    bc0 = b_c0.reshape(1, -1)
    bc1 = b_c1.reshape(1, -1)

    full = lambda s: pl.BlockSpec(s, lambda i: (0, 0))
    rows = lambda c: pl.BlockSpec((_RB, c), lambda i: (i, 0))
    col1 = pl.BlockSpec((_RB, 1), lambda i: (i, 0))

    emb, xw = pl.pallas_call(
        _k1_body,
        grid=(_NBLK,),
        in_specs=[rows(din), rows(din), full((din, din)), full((1, din)),
                  full((din, din)), full((1, din)), full((din, hid))],
        out_specs=[rows(din), rows(hid)],
        out_shape=[jax.ShapeDtypeStruct((n, din), f32),
                   jax.ShapeDtypeStruct((n, hid), f32)],
        interpret=_INTERPRET,
    )(features, x, W_g0, bg0, W_g1, bg1, W_c0)

    t, rs, cs = pl.pallas_call(
        _k2_body,
        grid=(_NBLK,),
        in_specs=[rows(din), full((n, din))],
        out_specs=[col1, col1, pl.BlockSpec((1, n), lambda i: (0, 0))],
        out_shape=[jax.ShapeDtypeStruct((n, 1), f32),
                   jax.ShapeDtypeStruct((n, 1), f32),
                   jax.ShapeDtypeStruct((1, n), f32)],
        scratch_shapes=[pltpu.VMEM((_RB, n), jnp.int32)],
        interpret=_INTERPRET,
    )(emb, emb)

    deg = (rs + cs.reshape(n, 1)) * 0.5
    d = 1.0 / (jnp.sqrt(deg) + 1e-10)
    t_row = t.reshape(1, n)
    d_row = d.reshape(1, n)

    h1w = pl.pallas_call(
        _k3_body,
        grid=(_NBLK,),
        in_specs=[rows(din), full((n, din)), col1, full((1, n)), col1,
                  full((1, n)), full((n, hid)), full((hid, dout)),
                  full((1, hid))],
        out_specs=rows(dout),
        out_shape=jax.ShapeDtypeStruct((n, dout), f32),
        interpret=_INTERPRET,
    )(emb, emb, t, t_row, d, d_row, xw, W_c1, bc0)

    adjn, out = pl.pallas_call(
        _k4_body,
        grid=(_NBLK,),
        in_specs=[rows(din), full((n, din)), col1, full((1, n)), col1,
                  full((1, n)), full((n, dout)), full((1, dout))],
        out_specs=[rows(n), rows(dout)],
        out_shape=[jax.ShapeDtypeStruct((n, n), f32),
                   jax.ShapeDtypeStruct((n, dout), f32)],
        interpret=_INTERPRET,
    )(emb, emb, t, t_row, d, d_row, h1w, bc1)

    return (out, adjn)
